# Initial kernel scaffold; baseline (speedup 1.0000x reference)
import jax
import jax.numpy as jnp
from jax.experimental import pallas as pl


def kernel(token_ids, weight):
    raise NotImplementedError("write your pallas kernel here")


# speedup vs baseline: 4.9452x; 4.9452x over previous
import functools

import jax
import jax.numpy as jnp
from jax import lax
from jax.experimental import pallas as pl
from jax.experimental.pallas import tpu as pltpu
from jax.experimental.pallas import tpu_sc as plsc

D = 32
NUM_CORES = 2
NUM_SUBCORES = 16
NUM_WORKERS = NUM_CORES * NUM_SUBCORES
CHUNK = 128
K = 16


@functools.partial(jax.jit, static_argnums=(2,))
def _embedding_lookup(idx2d, weight, total):
    n_per_w = total // NUM_WORKERS
    chunks_per_w = n_per_w // CHUNK
    outer = chunks_per_w // K
    mesh = plsc.VectorSubcoreMesh(core_axis_name="c", subcore_axis_name="s")

    @functools.partial(
        pl.kernel,
        mesh=mesh,
        out_type=jax.ShapeDtypeStruct((total, D), jnp.float32),
        scratch_types=[
            pltpu.VMEM((K, CHUNK), jnp.int32),
            pltpu.VMEM((K * CHUNK, D), jnp.float32),
            pltpu.SemaphoreType.DMA,
        ],
        compiler_params=pltpu.CompilerParams(use_tc_tiling_on_sc=False),
    )
    def k(idx_hbm, table_hbm, out_hbm, idx_v, rows_v, sem):
        wid = lax.axis_index("s") * NUM_CORES + lax.axis_index("c")
        row0 = wid * chunks_per_w

        def body(c, _):
            r = row0 + c * K
            pltpu.sync_copy(idx_hbm.at[pl.ds(r, K)], idx_v)
            copies = [
                pltpu.async_copy(
                    table_hbm.at[idx_v.at[j]],
                    rows_v.at[pl.ds(j * CHUNK, CHUNK)],
                    sem,
                )
                for j in range(K)
            ]
            for cp in copies:
                cp.wait()
            pltpu.sync_copy(rows_v, out_hbm.at[pl.ds(r * CHUNK, K * CHUNK)])
            return 0

        lax.fori_loop(0, outer, body, 0)

    return k(idx2d, weight)


def kernel(token_ids, weight):
    nb, nt = token_ids.shape
    total = nb * nt
    idx2d = token_ids.reshape(total // CHUNK, CHUNK).astype(jnp.int32)
    out = _embedding_lookup(idx2d, weight, total)
    return out.reshape(nb, nt, D)


# speedup vs baseline: 4.9507x; 1.0011x over previous
import functools

import jax
import jax.numpy as jnp
from jax import lax
from jax.experimental import pallas as pl
from jax.experimental.pallas import tpu as pltpu
from jax.experimental.pallas import tpu_sc as plsc

D = 32
NUM_CORES = 2
NUM_SUBCORES = 16
NUM_WORKERS = NUM_CORES * NUM_SUBCORES
CHUNK = 128
K = 8
NBUF = 2


@functools.partial(jax.jit, static_argnums=(2,))
def _embedding_lookup(idx2d, weight, total):
    n_per_w = total // NUM_WORKERS
    chunks_per_w = n_per_w // CHUNK
    n_sub = chunks_per_w // K
    assert n_sub % NBUF == 0
    mesh = plsc.VectorSubcoreMesh(core_axis_name="c", subcore_axis_name="s")

    @functools.partial(
        pl.kernel,
        mesh=mesh,
        out_type=jax.ShapeDtypeStruct((total, D), jnp.float32),
        scratch_types=[
            pltpu.VMEM((K, CHUNK), jnp.int32),
            pltpu.VMEM((K, CHUNK), jnp.int32),
            pltpu.VMEM((K * CHUNK, D), jnp.float32),
            pltpu.VMEM((K * CHUNK, D), jnp.float32),
            pltpu.SemaphoreType.DMA,
            pltpu.SemaphoreType.DMA,
        ],
        compiler_params=pltpu.CompilerParams(use_tc_tiling_on_sc=False),
    )
    def k(idx_hbm, table_hbm, out_hbm, idx_v0, idx_v1, rows_v0, rows_v1,
          sem0, sem1):
        idx_bufs = (idx_v0, idx_v1)
        rows_bufs = (rows_v0, rows_v1)
        sems = (sem0, sem1)
        wid = lax.axis_index("s") * NUM_CORES + lax.axis_index("c")
        row0 = wid * chunks_per_w

        def stage(c, b):
            r = row0 + c * K
            pltpu.sync_copy(idx_hbm.at[pl.ds(r, K)], idx_bufs[b])
            for j in range(K):
                pltpu.async_copy(
                    table_hbm.at[idx_bufs[b].at[j]],
                    rows_bufs[b].at[pl.ds(j * CHUNK, CHUNK)],
                    sems[b],
                )

        def wait_gathers(b):
            pltpu.make_async_copy(
                table_hbm.at[pl.ds(0, K * CHUNK)], rows_bufs[b], sems[b]
            ).wait()

        def flush(c, b):
            wait_gathers(b)
            pltpu.sync_copy(
                rows_bufs[b],
                out_hbm.at[pl.ds((row0 + c * K) * CHUNK, K * CHUNK)],
            )

        for b in range(NBUF):
            stage(b, b)

        @pl.loop(0, n_sub - NBUF, step=NBUF)
        def _(i):
            for b in range(NBUF):
                c = i + b
                flush(c, b)
                stage(c + NBUF, b)

        for b in range(NBUF):
            flush(n_sub - NBUF + b, b)

    return k(idx2d, weight)


def kernel(token_ids, weight):
    nb, nt = token_ids.shape
    total = nb * nt
    idx2d = token_ids.reshape(total // CHUNK, CHUNK).astype(jnp.int32)
    out = _embedding_lookup(idx2d, weight, total)
    return out.reshape(nb, nt, D)


# speedup vs baseline: 4.9805x; 1.0060x over previous
import functools

import jax
import jax.numpy as jnp
from jax import lax
from jax.experimental import pallas as pl
from jax.experimental.pallas import tpu as pltpu
from jax.experimental.pallas import tpu_sc as plsc

D = 32
NUM_CORES = 2
NUM_SUBCORES = 16
NUM_WORKERS = NUM_CORES * NUM_SUBCORES
RB = 8
NBUF = 2


@jax.jit
def _embedding_lookup(tokens, weight):
    nb, nt = tokens.shape
    splits = ((0, 96), (96, nt - 96))
    rows_per_w = nb // NUM_WORKERS
    n_sub = rows_per_w // RB
    assert n_sub % NBUF == 0 and all(s % 8 == 0 and w <= 128 for s, w in splits)
    mesh = plsc.VectorSubcoreMesh(core_axis_name="c", subcore_axis_name="s")

    @functools.partial(
        pl.kernel,
        mesh=mesh,
        out_type=jax.ShapeDtypeStruct((nb, nt, D), jnp.float32),
        scratch_types=[
            pltpu.VMEM((RB, nt), jnp.int32),
            pltpu.VMEM((RB, nt), jnp.int32),
            pltpu.VMEM((RB, nt, D), jnp.float32),
            pltpu.VMEM((RB, nt, D), jnp.float32),
            pltpu.SemaphoreType.DMA,
            pltpu.SemaphoreType.DMA,
        ],
        compiler_params=pltpu.CompilerParams(use_tc_tiling_on_sc=False),
    )
    def k(idx_hbm, table_hbm, out_hbm, idx_v0, idx_v1, rows_v0, rows_v1,
          sem0, sem1):
        idx_bufs = (idx_v0, idx_v1)
        rows_bufs = (rows_v0, rows_v1)
        sems = (sem0, sem1)
        wid = lax.axis_index("s") * NUM_CORES + lax.axis_index("c")
        row0 = wid * rows_per_w

        def stage(c, b):
            pltpu.sync_copy(idx_hbm.at[pl.ds(row0 + c * RB, RB)], idx_bufs[b])
            for i in range(RB):
                for s, w in splits:
                    pltpu.async_copy(
                        table_hbm.at[idx_bufs[b].at[i, pl.ds(s, w)]],
                        rows_bufs[b].at[i, pl.ds(s, w)],
                        sems[b],
                    )

        def flush(c, b):
            pltpu.make_async_copy(
                out_hbm.at[pl.ds(0, RB)], rows_bufs[b], sems[b]
            ).wait()
            pltpu.sync_copy(
                rows_bufs[b], out_hbm.at[pl.ds(row0 + c * RB, RB)]
            )

        for b in range(NBUF):
            stage(b, b)

        @pl.loop(0, n_sub - NBUF, step=NBUF)
        def _(i):
            for b in range(NBUF):
                flush(i + b, b)
                stage(i + b + NBUF, b)

        for b in range(NBUF):
            flush(n_sub - NBUF + b, b)

    return k(tokens, weight)


def kernel(token_ids, weight):
    return _embedding_lookup(token_ids.astype(jnp.int32), weight)
